# Initial kernel scaffold; baseline (speedup 1.0000x reference)
import jax
import jax.numpy as jnp
from jax.experimental import pallas as pl


def kernel(x, edge_index, edge_type, W_np, b_np, W_cp, b_cp, W_in, b_in, a_in, W_rel1, W_root1, b1, W_rel2, W_root2, b2, W_cls, b_cls):
    raise NotImplementedError("write your pallas kernel here")


# speedup vs baseline: 5.8792x; 5.8792x over previous
import functools

import jax
import jax.numpy as jnp
from jax import lax
from jax.experimental import pallas as pl
from jax.experimental.pallas import tpu as pltpu
from jax.experimental.pallas import tpu_sc as plsc

N = 10000
E = 320000
R = 2
D = 128
H = 64
NP_ = 10240
AROWS = 2 * NP_
NTILES = 16
NCORES = 2
EB = 128
NBLK = 160
G = 8
NGRP = NBLK // G
EPT = NBLK * EB
EP = EPT * NTILES
ROWS_PER_TILE = AROWS // NTILES
BM = 1024
GRID = NP_ // BM


def _sc_body(with_deg, *refs):
    if with_deg:
        (h_hbm, src_hbm, dst_hbm, typ_hbm, s_out, deg_out,
         acc, dega, sb0, db0, tb0, sb1, db1, tb1, rbuf0, rbuf1, c16,
         gsem0, gsem1, ssem0, ssem1) = refs
    else:
        (h_hbm, src_hbm, dst_hbm, typ_hbm, s_out,
         acc, sb0, db0, tb0, sb1, db1, tb1, rbuf0, rbuf1, c16,
         gsem0, gsem1, ssem0, ssem1) = refs
        dega = deg_out = None

    c = lax.axis_index("c")
    s = lax.axis_index("s")
    coff = c * NP_
    rbufs = (rbuf0, rbuf1)
    gsems = (gsem0, gsem1)
    sets = ((sb0, db0, tb0, ssem0), (sb1, db1, tb1, ssem1))

    def _stage(grp, si):
        sb, db, tb, sem = sets[si]
        off = grp * G
        pltpu.async_copy(src_hbm.at[s, pl.ds(off, G)], sb, sem)
        pltpu.async_copy(dst_hbm.at[s, pl.ds(off, G)], db, sem)
        pltpu.async_copy(typ_hbm.at[s, pl.ds(off, G)], tb, sem)

    def _stage_wait(si):
        sb, db, tb, sem = sets[si]
        dummy = src_hbm.at[s, pl.ds(0, G)]
        for b in (sb, db, tb):
            pltpu.make_async_copy(dummy, b, sem).wait()

    def _fixup(si):
        sb, db, tb, _ = sets[si]

        @pl.loop(0, G * (EB // 16))
        def _(k):
            r = k // (EB // 16)
            q = (k % (EB // 16)) * 16
            sb[r, pl.ds(q, 16)] = sb[r, pl.ds(q, 16)] + coff
            db[r, pl.ds(q, 16)] = db[r, pl.ds(q, 16)] + tb[r, pl.ds(q, 16)] * NP_

    z16 = jnp.zeros((16,), jnp.float32)

    @pl.loop(0, EB)
    def _fz(i):
        for q in range(H // 16):
            rbuf0[i, pl.ds(q * 16, 16)] = z16
        c16[i, pl.ds(0, 16)] = z16

    row0 = s * ROWS_PER_TILE

    @pl.loop(0, ROWS_PER_TILE // EB)
    def _zero(i):
        r0 = row0 + i * EB
        pltpu.sync_copy(rbuf0, acc.at[pl.ds(r0, EB)])
        if with_deg:
            @pl.when(c == 0)
            def _():
                pltpu.sync_copy(c16, dega.at[pl.ds(r0, EB)])

    if with_deg:
        o16 = jnp.ones((16,), jnp.float32)

        @pl.loop(0, EB)
        def _fo(i):
            c16[i, pl.ds(0, 16)] = o16

    _stage(0, 0)
    plsc.subcore_barrier()

    def _gwait(bi):
        pltpu.make_async_copy(h_hbm.at[pl.ds(0, EB)], rbufs[bi], gsems[bi]).wait()

    def _group(grp, si, stage_next):
        sb, db, tb, _ = sets[si]
        _stage_wait(si)
        _fixup(si)
        if stage_next is not None:
            @pl.when(stage_next)
            def _():
                _stage(grp + 1, 1 - si)
        pltpu.async_copy(h_hbm.at[sb.at[0]], rbufs[0], gsems[0])
        for j in range(G):
            bi = j % 2
            if j + 1 < G:
                pltpu.async_copy(h_hbm.at[sb.at[j + 1]], rbufs[1 - bi],
                                 gsems[1 - bi])
            _gwait(bi)
            pltpu.sync_copy(rbufs[bi], acc.at[db.at[j]], add=True)
            if with_deg:
                @pl.when(c == 0)
                def _():
                    pltpu.sync_copy(c16, dega.at[db.at[j]], add=True)

    @pl.loop(0, NGRP // 2)
    def _pair(i):
        _group(2 * i, 0, True)
        _group(2 * i + 1, 1, 2 * i + 2 < NGRP)

    plsc.subcore_barrier()

    @pl.loop(0, ROWS_PER_TILE // EB)
    def _out(i):
        r0 = row0 + i * EB
        pltpu.sync_copy(acc.at[pl.ds(r0, EB)], rbuf0)
        pltpu.sync_copy(rbuf0, s_out.at[c, pl.ds(r0, EB)])
        if with_deg:
            @pl.when(c == 0)
            def _():
                pltpu.sync_copy(dega.at[pl.ds(r0, EB)], c16)
                pltpu.sync_copy(c16, deg_out.at[pl.ds(r0, EB)])


def _make_sc_layer(with_deg):
    out_type = [jax.ShapeDtypeStruct((NCORES, AROWS, H), jnp.float32)]
    scratch = [
        pltpu.VMEM_SHARED((AROWS, H), jnp.float32),
    ]
    if with_deg:
        out_type.append(jax.ShapeDtypeStruct((AROWS, 16), jnp.float32))
        scratch.append(pltpu.VMEM_SHARED((AROWS, 16), jnp.float32))
    scratch += [
        pltpu.VMEM((G, EB), jnp.int32),
        pltpu.VMEM((G, EB), jnp.int32),
        pltpu.VMEM((G, EB), jnp.int32),
        pltpu.VMEM((G, EB), jnp.int32),
        pltpu.VMEM((G, EB), jnp.int32),
        pltpu.VMEM((G, EB), jnp.int32),
        pltpu.VMEM((EB, H), jnp.float32),
        pltpu.VMEM((EB, H), jnp.float32),
        pltpu.VMEM((EB, 16), jnp.float32),
        pltpu.SemaphoreType.DMA,
        pltpu.SemaphoreType.DMA,
        pltpu.SemaphoreType.DMA,
        pltpu.SemaphoreType.DMA,
    ]
    mesh = plsc.VectorSubcoreMesh(core_axis_name="c", subcore_axis_name="s",
                                  num_cores=NCORES, num_subcores=NTILES)
    return pl.kernel(
        functools.partial(_sc_body, with_deg),
        out_type=tuple(out_type) if with_deg else out_type[0],
        mesh=mesh,
        scratch_types=scratch,
        compiler_params=pltpu.CompilerParams(use_tc_tiling_on_sc=False),
    )


_DOT = functools.partial(jnp.dot, preferred_element_type=jnp.float32,
                         precision=jax.lax.Precision.HIGHEST)


def _encode_body(x_ref, wenc_ref, benc_ref, win_ref, bin_ref, ain_ref, out_ref):
    h = _DOT(x_ref[...], wenc_ref[...]) + benc_ref[...]
    h = jnp.where(h >= 0, h, 0.01 * h)
    h = _DOT(h, win_ref[...]) + bin_ref[...]
    h = jnp.where(h >= 0, h, ain_ref[...] * h)
    out_ref[0] = h[:, :H]
    out_ref[1] = h[:, H:]


def _combine_body(h_ref, s0_ref, s1_ref, d0_ref, d1_ref, wroot_ref, wrel_ref,
                  b_ref, wcls_ref, bcls_ref, out_ref, *, final):
    h = jnp.concatenate([h_ref[0], h_ref[1]], axis=1)
    s0 = jnp.concatenate([s0_ref[0], s0_ref[1]], axis=1)
    s1 = jnp.concatenate([s1_ref[0], s1_ref[1]], axis=1)
    inv0 = 1.0 / jnp.maximum(d0_ref[...][:, :1], 1.0)
    inv1 = 1.0 / jnp.maximum(d1_ref[...][:, :1], 1.0)
    o = (_DOT(h, wroot_ref[...]) + b_ref[...]
         + _DOT(s0 * inv0, wrel_ref[0]) + _DOT(s1 * inv1, wrel_ref[1]))
    if final:
        out_ref[...] = _DOT(o, wcls_ref[...]) + bcls_ref[...]
    else:
        out_ref[0] = o[:, :H]
        out_ref[1] = o[:, H:]


def _full(shape):
    return pl.BlockSpec(shape, lambda i: (0,) * len(shape))


_CAT_SPEC = pl.BlockSpec((2, BM, H), lambda i: (0, i, 0))


def _encode_call(x_p, wenc, benc, win, bin_, ain):
    return pl.pallas_call(
        _encode_body,
        grid=(GRID,),
        in_specs=[
            pl.BlockSpec((BM, 8), lambda i: (i, 0)),
            _full((8, D)), _full((1, D)), _full((D, D)), _full((1, D)),
            _full((1, D)),
        ],
        out_specs=_CAT_SPEC,
        out_shape=jax.ShapeDtypeStruct((2, NP_, H), jnp.float32),
    )(x_p, wenc, benc, win, bin_, ain)


def _combine_call(final, hcat, s_acc, deg, wroot, wrel, b, wcls, bcls):
    out_shape = (jax.ShapeDtypeStruct((NP_, D), jnp.float32) if final
                 else jax.ShapeDtypeStruct((2, NP_, H), jnp.float32))
    out_specs = (pl.BlockSpec((BM, D), lambda i: (i, 0)) if final
                 else _CAT_SPEC)
    s_blk = lambda r: pl.BlockSpec((2, BM, H),
                                   lambda i, r=r: (0, r * (NP_ // BM) + i, 0))
    d_blk = lambda r: pl.BlockSpec((BM, 16),
                                   lambda i, r=r: (r * (NP_ // BM) + i, 0))
    return pl.pallas_call(
        functools.partial(_combine_body, final=final),
        grid=(GRID,),
        in_specs=[
            _CAT_SPEC,
            s_blk(0), s_blk(1),
            d_blk(0), d_blk(1),
            _full((D, D)), _full((R, D, D)), _full((1, D)),
            _full((D, D)), _full((1, D)),
        ],
        out_specs=out_specs,
        out_shape=out_shape,
    )(hcat, s_acc, s_acc, deg, deg, wroot, wrel, b, wcls, bcls)


_sc_cache = {}


def _sc_layer(with_deg, *args):
    if with_deg not in _sc_cache:
        _sc_cache[with_deg] = _make_sc_layer(with_deg)
    return _sc_cache[with_deg](*args)


def kernel(x, edge_index, edge_type, W_np, b_np, W_cp, b_cp, W_in, b_in, a_in,
           W_rel1, W_root1, b1, W_rel2, W_root2, b2, W_cls, b_cls):
    f32 = jnp.float32
    x_p = jnp.pad(x, ((0, NP_ - N), (0, 0)))
    wenc = jnp.zeros((8, D), f32).at[:5, :H].set(W_np).at[5:, H:].set(W_cp)
    benc = jnp.concatenate([b_np, b_cp]).reshape(1, D)
    src = jnp.pad(edge_index[0], (0, EP - E)).reshape(NTILES, NBLK, EB)
    dst = jnp.pad(edge_index[1], (0, EP - E),
                  constant_values=N).reshape(NTILES, NBLK, EB)
    typ = jnp.pad(edge_type, (0, EP - E)).reshape(NTILES, NBLK, EB)

    h0cat = _encode_call(x_p, wenc, benc, W_in, b_in.reshape(1, D),
                         a_in.reshape(1, D))

    s1, deg = _sc_layer(True, h0cat.reshape(NCORES * NP_, H), src, dst, typ)
    h1cat = _combine_call(False, h0cat, s1, deg, W_root1, W_rel1,
                          b1.reshape(1, D), W_cls, b_cls.reshape(1, D))

    s2 = _sc_layer(False, h1cat.reshape(NCORES * NP_, H), src, dst, typ)
    out = _combine_call(True, h1cat, s2, deg, W_root2, W_rel2,
                        b2.reshape(1, D), W_cls, b_cls.reshape(1, D))
    return out[:N]


# speedup vs baseline: 6.0183x; 1.0237x over previous
import functools

import jax
import jax.numpy as jnp
from jax import lax
from jax.experimental import pallas as pl
from jax.experimental.pallas import tpu as pltpu
from jax.experimental.pallas import tpu_sc as plsc

N = 10000
E = 320000
R = 2
D = 128
H = 64
NP_ = 10240
AROWS = 2 * NP_
NTILES = 16
NCORES = 2
EB = 128
NBLK = 160
G = 8
NGRP = NBLK // G
EPT = NBLK * EB
EP = EPT * NTILES
ROWS_PER_TILE = AROWS // NTILES
BM = 1024
GRID = NP_ // BM


def _sc_body(with_deg, *refs):
    if with_deg:
        (h_hbm, src_hbm, dst_hbm, typ_hbm, s_out, deg_out,
         acc, dega, sb0, db0, tb0, sb1, db1, tb1, rbuf0, rbuf1, c16,
         gsem0, gsem1, ssem0, ssem1, wsem0, wsem1) = refs
    else:
        (h_hbm, src_hbm, dst_hbm, typ_hbm, s_out,
         acc, sb0, db0, tb0, sb1, db1, tb1, rbuf0, rbuf1, c16,
         gsem0, gsem1, ssem0, ssem1, wsem0, wsem1) = refs
        dega = deg_out = None

    c = lax.axis_index("c")
    s = lax.axis_index("s")
    coff = c * NP_
    rbufs = (rbuf0, rbuf1)
    gsems = (gsem0, gsem1)
    wsems = (wsem0, wsem1)
    sets = ((sb0, db0, tb0, ssem0), (sb1, db1, tb1, ssem1))
    hd_dummy = deg_out.at[pl.ds(0, EB)] if with_deg else None

    def _stage(grp, si):
        sb, db, tb, sem = sets[si]
        off = grp * G
        pltpu.async_copy(src_hbm.at[s, pl.ds(off, G)], sb, sem)
        pltpu.async_copy(dst_hbm.at[s, pl.ds(off, G)], db, sem)
        pltpu.async_copy(typ_hbm.at[s, pl.ds(off, G)], tb, sem)

    def _stage_wait(si):
        sb, db, tb, sem = sets[si]
        dummy = src_hbm.at[s, pl.ds(0, G)]
        for b in (sb, db, tb):
            pltpu.make_async_copy(dummy, b, sem).wait()

    def _fixup(si):
        sb, db, tb, _ = sets[si]

        @pl.loop(0, G * (EB // 16))
        def _(k):
            r = k // (EB // 16)
            q = (k % (EB // 16)) * 16
            sb[r, pl.ds(q, 16)] = sb[r, pl.ds(q, 16)] + coff
            db[r, pl.ds(q, 16)] = db[r, pl.ds(q, 16)] + tb[r, pl.ds(q, 16)] * NP_

    z16 = jnp.zeros((16,), jnp.float32)

    @pl.loop(0, EB)
    def _fz(i):
        for q in range(H // 16):
            rbuf0[i, pl.ds(q * 16, 16)] = z16
        c16[i, pl.ds(0, 16)] = z16

    row0 = s * ROWS_PER_TILE

    @pl.loop(0, ROWS_PER_TILE // EB)
    def _zero(i):
        r0 = row0 + i * EB
        pltpu.sync_copy(rbuf0, acc.at[pl.ds(r0, EB)])
        if with_deg:
            @pl.when(c == 0)
            def _():
                pltpu.sync_copy(c16, dega.at[pl.ds(r0, EB)])

    if with_deg:
        o16 = jnp.ones((16,), jnp.float32)

        @pl.loop(0, EB)
        def _fo(i):
            c16[i, pl.ds(0, 16)] = o16

    _stage(0, 0)
    plsc.subcore_barrier()

    def _gwait(bi):
        pltpu.make_async_copy(h_hbm.at[pl.ds(0, EB)], rbufs[bi], gsems[bi]).wait()

    def _wwait(bi):
        pltpu.make_async_copy(h_hbm.at[pl.ds(0, EB)], rbufs[bi], wsems[bi]).wait()
        if with_deg:
            @pl.when(c == 0)
            def _():
                pltpu.make_async_copy(hd_dummy, c16, wsems[bi]).wait()

    def _group(grp, si, stage_next, first, last):
        sb, db, tb, _ = sets[si]
        _stage_wait(si)
        _fixup(si)
        if stage_next is not None:
            @pl.when(stage_next)
            def _():
                _stage(grp + 1, 1 - si)
        pltpu.async_copy(h_hbm.at[sb.at[0]], rbufs[0], gsems[0])
        for j in range(G):
            bi = j % 2
            if j + 1 < G:
                if not (first and j == 0):
                    _wwait(1 - bi)
                pltpu.async_copy(h_hbm.at[sb.at[j + 1]], rbufs[1 - bi],
                                 gsems[1 - bi])
            _gwait(bi)
            pltpu.async_copy(rbufs[bi], acc.at[db.at[j]], wsems[bi], add=True)
            if with_deg:
                @pl.when(c == 0)
                def _():
                    pltpu.async_copy(c16, dega.at[db.at[j]], wsems[bi],
                                     add=True)
        if last:
            _wwait(0)
            _wwait(1)
        else:
            _wwait(0)

    _group(0, 0, True, True, False)

    @pl.loop(0, NGRP // 2 - 1)
    def _pair(i):
        _group(2 * i + 1, 1, True, False, False)
        _group(2 * i + 2, 0, True, False, False)

    _group(NGRP - 1, 1, None, False, True)

    plsc.subcore_barrier()

    @pl.loop(0, ROWS_PER_TILE // EB)
    def _out(i):
        r0 = row0 + i * EB
        pltpu.sync_copy(acc.at[pl.ds(r0, EB)], rbuf0)
        pltpu.sync_copy(rbuf0, s_out.at[c, pl.ds(r0, EB)])
        if with_deg:
            @pl.when(c == 0)
            def _():
                pltpu.sync_copy(dega.at[pl.ds(r0, EB)], c16)
                pltpu.sync_copy(c16, deg_out.at[pl.ds(r0, EB)])


def _make_sc_layer(with_deg):
    out_type = [jax.ShapeDtypeStruct((NCORES, AROWS, H), jnp.float32)]
    scratch = [
        pltpu.VMEM_SHARED((AROWS, H), jnp.float32),
    ]
    if with_deg:
        out_type.append(jax.ShapeDtypeStruct((AROWS, 16), jnp.float32))
        scratch.append(pltpu.VMEM_SHARED((AROWS, 16), jnp.float32))
    scratch += [
        pltpu.VMEM((G, EB), jnp.int32),
        pltpu.VMEM((G, EB), jnp.int32),
        pltpu.VMEM((G, EB), jnp.int32),
        pltpu.VMEM((G, EB), jnp.int32),
        pltpu.VMEM((G, EB), jnp.int32),
        pltpu.VMEM((G, EB), jnp.int32),
        pltpu.VMEM((EB, H), jnp.float32),
        pltpu.VMEM((EB, H), jnp.float32),
        pltpu.VMEM((EB, 16), jnp.float32),
        pltpu.SemaphoreType.DMA,
        pltpu.SemaphoreType.DMA,
        pltpu.SemaphoreType.DMA,
        pltpu.SemaphoreType.DMA,
        pltpu.SemaphoreType.DMA,
        pltpu.SemaphoreType.DMA,
    ]
    mesh = plsc.VectorSubcoreMesh(core_axis_name="c", subcore_axis_name="s",
                                  num_cores=NCORES, num_subcores=NTILES)
    return pl.kernel(
        functools.partial(_sc_body, with_deg),
        out_type=tuple(out_type) if with_deg else out_type[0],
        mesh=mesh,
        scratch_types=scratch,
        compiler_params=pltpu.CompilerParams(use_tc_tiling_on_sc=False),
    )


_DOT = functools.partial(jnp.dot, preferred_element_type=jnp.float32,
                         precision=jax.lax.Precision.HIGHEST)


def _encode_body(x_ref, wenc_ref, benc_ref, win_ref, bin_ref, ain_ref, out_ref):
    h = _DOT(x_ref[...], wenc_ref[...]) + benc_ref[...]
    h = jnp.where(h >= 0, h, 0.01 * h)
    h = _DOT(h, win_ref[...]) + bin_ref[...]
    h = jnp.where(h >= 0, h, ain_ref[...] * h)
    out_ref[0] = h[:, :H]
    out_ref[1] = h[:, H:]


def _combine_body(h_ref, s0_ref, s1_ref, d0_ref, d1_ref, wroot_ref, wrel_ref,
                  b_ref, wcls_ref, bcls_ref, out_ref, *, final):
    h = jnp.concatenate([h_ref[0], h_ref[1]], axis=1)
    s0 = jnp.concatenate([s0_ref[0], s0_ref[1]], axis=1)
    s1 = jnp.concatenate([s1_ref[0], s1_ref[1]], axis=1)
    inv0 = 1.0 / jnp.maximum(d0_ref[...][:, :1], 1.0)
    inv1 = 1.0 / jnp.maximum(d1_ref[...][:, :1], 1.0)
    o = (_DOT(h, wroot_ref[...]) + b_ref[...]
         + _DOT(s0 * inv0, wrel_ref[0]) + _DOT(s1 * inv1, wrel_ref[1]))
    if final:
        out_ref[...] = _DOT(o, wcls_ref[...]) + bcls_ref[...]
    else:
        out_ref[0] = o[:, :H]
        out_ref[1] = o[:, H:]


def _full(shape):
    return pl.BlockSpec(shape, lambda i: (0,) * len(shape))


_CAT_SPEC = pl.BlockSpec((2, BM, H), lambda i: (0, i, 0))


def _encode_call(x_p, wenc, benc, win, bin_, ain):
    return pl.pallas_call(
        _encode_body,
        grid=(GRID,),
        in_specs=[
            pl.BlockSpec((BM, 8), lambda i: (i, 0)),
            _full((8, D)), _full((1, D)), _full((D, D)), _full((1, D)),
            _full((1, D)),
        ],
        out_specs=_CAT_SPEC,
        out_shape=jax.ShapeDtypeStruct((2, NP_, H), jnp.float32),
    )(x_p, wenc, benc, win, bin_, ain)


def _combine_call(final, hcat, s_acc, deg, wroot, wrel, b, wcls, bcls):
    out_shape = (jax.ShapeDtypeStruct((NP_, D), jnp.float32) if final
                 else jax.ShapeDtypeStruct((2, NP_, H), jnp.float32))
    out_specs = (pl.BlockSpec((BM, D), lambda i: (i, 0)) if final
                 else _CAT_SPEC)
    s_blk = lambda r: pl.BlockSpec((2, BM, H),
                                   lambda i, r=r: (0, r * (NP_ // BM) + i, 0))
    d_blk = lambda r: pl.BlockSpec((BM, 16),
                                   lambda i, r=r: (r * (NP_ // BM) + i, 0))
    return pl.pallas_call(
        functools.partial(_combine_body, final=final),
        grid=(GRID,),
        in_specs=[
            _CAT_SPEC,
            s_blk(0), s_blk(1),
            d_blk(0), d_blk(1),
            _full((D, D)), _full((R, D, D)), _full((1, D)),
            _full((D, D)), _full((1, D)),
        ],
        out_specs=out_specs,
        out_shape=out_shape,
    )(hcat, s_acc, s_acc, deg, deg, wroot, wrel, b, wcls, bcls)


_sc_cache = {}


def _sc_layer(with_deg, *args):
    if with_deg not in _sc_cache:
        _sc_cache[with_deg] = _make_sc_layer(with_deg)
    return _sc_cache[with_deg](*args)


def kernel(x, edge_index, edge_type, W_np, b_np, W_cp, b_cp, W_in, b_in, a_in,
           W_rel1, W_root1, b1, W_rel2, W_root2, b2, W_cls, b_cls):
    f32 = jnp.float32
    x_p = jnp.pad(x, ((0, NP_ - N), (0, 0)))
    wenc = jnp.zeros((8, D), f32).at[:5, :H].set(W_np).at[5:, H:].set(W_cp)
    benc = jnp.concatenate([b_np, b_cp]).reshape(1, D)
    src = jnp.pad(edge_index[0], (0, EP - E)).reshape(NTILES, NBLK, EB)
    dst = jnp.pad(edge_index[1], (0, EP - E),
                  constant_values=N).reshape(NTILES, NBLK, EB)
    typ = jnp.pad(edge_type, (0, EP - E)).reshape(NTILES, NBLK, EB)

    h0cat = _encode_call(x_p, wenc, benc, W_in, b_in.reshape(1, D),
                         a_in.reshape(1, D))

    s1, deg = _sc_layer(True, h0cat.reshape(NCORES * NP_, H), src, dst, typ)
    h1cat = _combine_call(False, h0cat, s1, deg, W_root1, W_rel1,
                          b1.reshape(1, D), W_cls, b_cls.reshape(1, D))

    s2 = _sc_layer(False, h1cat.reshape(NCORES * NP_, H), src, dst, typ)
    out = _combine_call(True, h1cat, s2, deg, W_root2, W_rel2,
                        b2.reshape(1, D), W_cls, b_cls.reshape(1, D))
    return out[:N]


# speedup vs baseline: 10.9829x; 1.8249x over previous
import functools

import jax
import jax.numpy as jnp
from jax import lax
from jax.experimental import pallas as pl
from jax.experimental.pallas import tpu as pltpu
from jax.experimental.pallas import tpu_sc as plsc

N = 10000
E = 320000
R = 2
D = 128
H = 64
NP_ = 10016
AROWS = 2 * NP_
NTILES = 16
NCORES = 2
EB = 64
G = 8
NBLK = 320
NGRP = NBLK // G
EPT = NBLK * EB
EP = EPT * NTILES
RPT = AROWS // NTILES
HPT = NP_ // NTILES
BM = 1024
GRID = (NP_ + BM - 1) // BM


def _prep_body(dst_hbm, typ_hbm, z16_hbm, sidx_out, deg_out,
               dega, db0, tb0, ones, ssem, osem, wsem):
    c = lax.axis_index("c")
    s = lax.axis_index("s")
    row0 = s * RPT

    def _stage(grp):
        off = grp * G
        pltpu.async_copy(dst_hbm.at[s, pl.ds(off, G)], db0, ssem)
        pltpu.async_copy(typ_hbm.at[s, pl.ds(off, G)], tb0, ssem)

    def _stage_wait():
        dummy = dst_hbm.at[s, pl.ds(0, G)]
        pltpu.make_async_copy(dummy, db0, ssem).wait()
        pltpu.make_async_copy(dummy, tb0, ssem).wait()

    @pl.when(c == 0)
    def _():
        o16 = jnp.ones((16,), jnp.float32)

        @pl.loop(0, EB)
        def _fo(i):
            ones[i, pl.ds(0, 16)] = o16

        pltpu.sync_copy(z16_hbm, dega.at[pl.ds(row0, RPT)])
        _stage(0)

    plsc.subcore_barrier()

    @pl.when(c == 0)
    def _():
        @pl.loop(0, NGRP)
        def _g(grp):
            _stage_wait()

            @pl.loop(0, G * (EB // 16))
            def _(k):
                r = k // (EB // 16)
                q = (k % (EB // 16)) * 16
                db0[r, pl.ds(q, 16)] = (db0[r, pl.ds(q, 16)]
                                        + tb0[r, pl.ds(q, 16)] * NP_)

            pltpu.async_copy(db0, sidx_out.at[s, pl.ds(grp * G, G)], osem)
            for j in range(G):
                pltpu.async_copy(ones, dega.at[db0.at[j]], wsem, add=True)
            for j in range(G):
                pltpu.make_async_copy(z16_hbm.at[pl.ds(0, EB)], ones,
                                      wsem).wait()
            pltpu.make_async_copy(dst_hbm.at[s, pl.ds(0, G)], db0, osem).wait()

            @pl.when(grp + 1 < NGRP)
            def _():
                _stage(grp + 1)

    plsc.subcore_barrier()

    @pl.when(c == 0)
    def _():
        pltpu.sync_copy(
            dega.at[pl.ds(row0, RPT)],
            deg_out.at[s // (NTILES // R), pl.ds((s % (NTILES // R)) * RPT, RPT)])


def _make_prep():
    mesh = plsc.VectorSubcoreMesh(core_axis_name="c", subcore_axis_name="s",
                                  num_cores=NCORES, num_subcores=NTILES)
    return pl.kernel(
        _prep_body,
        out_type=(jax.ShapeDtypeStruct((NTILES, NBLK, EB), jnp.int32),
                  jax.ShapeDtypeStruct((R, NP_, 16), jnp.float32)),
        mesh=mesh,
        scratch_types=[
            pltpu.VMEM_SHARED((AROWS, 16), jnp.float32),
            pltpu.VMEM((G, EB), jnp.int32),
            pltpu.VMEM((G, EB), jnp.int32),
            pltpu.VMEM((EB, 16), jnp.float32),
            pltpu.SemaphoreType.DMA,
            pltpu.SemaphoreType.DMA,
            pltpu.SemaphoreType.DMA,
        ],
        compiler_params=pltpu.CompilerParams(use_tc_tiling_on_sc=False),
    )


def _layer_body(h_hbm, src_hbm, sidx_hbm, z64_hbm, s_out,
                acc, htab, sb0, db0, sb1, db1, rbuf0, rbuf1,
                gsem0, gsem1, ssem0, ssem1, wsem0, wsem1):
    c = lax.axis_index("c")
    s = lax.axis_index("s")
    rbufs = (rbuf0, rbuf1)
    gsems = (gsem0, gsem1)
    wsems = (wsem0, wsem1)
    sets = ((sb0, db0, ssem0), (sb1, db1, ssem1))

    def _stage(grp, si):
        sb, db, sem = sets[si]
        off = grp * G
        pltpu.async_copy(src_hbm.at[s, pl.ds(off, G)], sb, sem)
        pltpu.async_copy(sidx_hbm.at[s, pl.ds(off, G)], db, sem)

    def _stage_wait(si):
        sb, db, sem = sets[si]
        dummy = src_hbm.at[s, pl.ds(0, G)]
        pltpu.make_async_copy(dummy, sb, sem).wait()
        pltpu.make_async_copy(dummy, db, sem).wait()

    hrow0 = s * HPT
    pltpu.sync_copy(h_hbm.at[pl.ds(c * NP_ + hrow0, HPT)],
                    htab.at[pl.ds(hrow0, HPT)])
    row0 = s * RPT
    pltpu.sync_copy(z64_hbm, acc.at[pl.ds(row0, RPT)])
    _stage(0, 0)
    plsc.subcore_barrier()

    def _gwait(bi):
        pltpu.make_async_copy(h_hbm.at[pl.ds(0, EB)], rbufs[bi],
                              gsems[bi]).wait()

    def _wwait(bi):
        pltpu.make_async_copy(h_hbm.at[pl.ds(0, EB)], rbufs[bi],
                              wsems[bi]).wait()

    def _group(grp, si, stage_next, first, last):
        sb, db, _ = sets[si]
        _stage_wait(si)
        if stage_next:
            @pl.when(grp + 1 < NGRP)
            def _():
                _stage(grp + 1, 1 - si)
        pltpu.async_copy(htab.at[sb.at[0]], rbufs[0], gsems[0])
        for j in range(G):
            bi = j % 2
            if j + 1 < G:
                if not (first and j == 0):
                    _wwait(1 - bi)
                pltpu.async_copy(htab.at[sb.at[j + 1]], rbufs[1 - bi],
                                 gsems[1 - bi])
            _gwait(bi)
            pltpu.async_copy(rbufs[bi], acc.at[db.at[j]], wsems[bi], add=True)
        if last:
            _wwait(0)
            _wwait(1)
        else:
            _wwait(0)

    _group(0, 0, True, True, False)

    @pl.loop(0, NGRP // 2 - 1)
    def _pair(i):
        _group(2 * i + 1, 1, True, False, False)
        _group(2 * i + 2, 0, True, False, False)

    _group(NGRP - 1, 1, False, False, True)

    plsc.subcore_barrier()

    pltpu.sync_copy(
        acc.at[pl.ds(row0, RPT)],
        s_out.at[c, s // (NTILES // R), pl.ds((s % (NTILES // R)) * RPT, RPT)])


def _make_layer():
    mesh = plsc.VectorSubcoreMesh(core_axis_name="c", subcore_axis_name="s",
                                  num_cores=NCORES, num_subcores=NTILES)
    return pl.kernel(
        _layer_body,
        out_type=jax.ShapeDtypeStruct((NCORES, R, NP_, H), jnp.float32),
        mesh=mesh,
        scratch_types=[
            pltpu.VMEM_SHARED((AROWS, H), jnp.float32),
            pltpu.VMEM_SHARED((NP_, H), jnp.float32),
            pltpu.VMEM((G, EB), jnp.int32),
            pltpu.VMEM((G, EB), jnp.int32),
            pltpu.VMEM((G, EB), jnp.int32),
            pltpu.VMEM((G, EB), jnp.int32),
            pltpu.VMEM((EB, H), jnp.float32),
            pltpu.VMEM((EB, H), jnp.float32),
            pltpu.SemaphoreType.DMA,
            pltpu.SemaphoreType.DMA,
            pltpu.SemaphoreType.DMA,
            pltpu.SemaphoreType.DMA,
            pltpu.SemaphoreType.DMA,
            pltpu.SemaphoreType.DMA,
        ],
        compiler_params=pltpu.CompilerParams(use_tc_tiling_on_sc=False),
    )


_DOT = functools.partial(jnp.dot, preferred_element_type=jnp.float32,
                         precision=jax.lax.Precision.HIGHEST)


def _encode_body(x_ref, wenc_ref, benc_ref, win_ref, bin_ref, ain_ref, out_ref):
    h = _DOT(x_ref[...], wenc_ref[...]) + benc_ref[...]
    h = jnp.where(h >= 0, h, 0.01 * h)
    h = _DOT(h, win_ref[...]) + bin_ref[...]
    h = jnp.where(h >= 0, h, ain_ref[...] * h)
    out_ref[0] = h[:, :H]
    out_ref[1] = h[:, H:]


def _combine_body(h_ref, s_ref, d_ref, wroot_ref, wrel_ref,
                  b_ref, wcls_ref, bcls_ref, out_ref, *, final):
    h = jnp.concatenate([h_ref[0], h_ref[1]], axis=1)
    s0 = jnp.concatenate([s_ref[0, 0], s_ref[1, 0]], axis=1)
    s1 = jnp.concatenate([s_ref[0, 1], s_ref[1, 1]], axis=1)
    inv0 = 1.0 / jnp.maximum(d_ref[0][:, :1], 1.0)
    inv1 = 1.0 / jnp.maximum(d_ref[1][:, :1], 1.0)
    o = (_DOT(h, wroot_ref[...]) + b_ref[...]
         + _DOT(s0 * inv0, wrel_ref[0]) + _DOT(s1 * inv1, wrel_ref[1]))
    if final:
        out_ref[...] = _DOT(o, wcls_ref[...]) + bcls_ref[...]
    else:
        out_ref[0] = o[:, :H]
        out_ref[1] = o[:, H:]


def _full(shape):
    return pl.BlockSpec(shape, lambda i: (0,) * len(shape))


_CAT_SPEC = pl.BlockSpec((2, BM, H), lambda i: (0, i, 0))


def _encode_call(x_p, wenc, benc, win, bin_, ain):
    return pl.pallas_call(
        _encode_body,
        grid=(GRID,),
        in_specs=[
            pl.BlockSpec((BM, 8), lambda i: (i, 0)),
            _full((8, D)), _full((1, D)), _full((D, D)), _full((1, D)),
            _full((1, D)),
        ],
        out_specs=_CAT_SPEC,
        out_shape=jax.ShapeDtypeStruct((2, NP_, H), jnp.float32),
    )(x_p, wenc, benc, win, bin_, ain)


def _combine_call(final, hcat, s_acc, deg, wroot, wrel, b, wcls, bcls):
    out_shape = (jax.ShapeDtypeStruct((NP_, D), jnp.float32) if final
                 else jax.ShapeDtypeStruct((2, NP_, H), jnp.float32))
    out_specs = (pl.BlockSpec((BM, D), lambda i: (i, 0)) if final
                 else _CAT_SPEC)
    return pl.pallas_call(
        functools.partial(_combine_body, final=final),
        grid=(GRID,),
        in_specs=[
            _CAT_SPEC,
            pl.BlockSpec((2, 2, BM, H), lambda i: (0, 0, i, 0)),
            pl.BlockSpec((2, BM, 16), lambda i: (0, i, 0)),
            _full((D, D)), _full((R, D, D)), _full((1, D)),
            _full((D, D)), _full((1, D)),
        ],
        out_specs=out_specs,
        out_shape=out_shape,
    )(hcat, s_acc, deg, wroot, wrel, b, wcls, bcls)


_sc_cache = {}


def _sc_prep(*args):
    if "prep" not in _sc_cache:
        _sc_cache["prep"] = _make_prep()
    return _sc_cache["prep"](*args)


def _sc_layer(*args):
    if "layer" not in _sc_cache:
        _sc_cache["layer"] = _make_layer()
    return _sc_cache["layer"](*args)


def kernel(x, edge_index, edge_type, W_np, b_np, W_cp, b_cp, W_in, b_in, a_in,
           W_rel1, W_root1, b1, W_rel2, W_root2, b2, W_cls, b_cls):
    f32 = jnp.float32
    x_p = jnp.pad(x, ((0, NP_ - N), (0, 0)))
    wenc = jnp.zeros((8, D), f32).at[:5, :H].set(W_np).at[5:, H:].set(W_cp)
    benc = jnp.concatenate([b_np, b_cp]).reshape(1, D)
    src = jnp.pad(edge_index[0], (0, EP - E)).reshape(NTILES, NBLK, EB)
    dst = jnp.pad(edge_index[1], (0, EP - E),
                  constant_values=N).reshape(NTILES, NBLK, EB)
    typ = jnp.pad(edge_type, (0, EP - E)).reshape(NTILES, NBLK, EB)
    z16 = jnp.zeros((RPT, 16), f32)
    z64 = jnp.zeros((RPT, H), f32)

    sidx, deg = _sc_prep(dst, typ, z16)

    h0cat = _encode_call(x_p, wenc, benc, W_in, b_in.reshape(1, D),
                         a_in.reshape(1, D))

    s1 = _sc_layer(h0cat.reshape(NCORES * NP_, H), src, sidx, z64)
    h1cat = _combine_call(False, h0cat, s1, deg, W_root1, W_rel1,
                          b1.reshape(1, D), W_cls, b_cls.reshape(1, D))

    s2 = _sc_layer(h1cat.reshape(NCORES * NP_, H), src, sidx, z64)
    out = _combine_call(True, h1cat, s2, deg, W_root2, W_rel2,
                        b2.reshape(1, D), W_cls, b_cls.reshape(1, D))
    return out[:N]


# speedup vs baseline: 11.3697x; 1.0352x over previous
import functools

import jax
import jax.numpy as jnp
from jax import lax
from jax.experimental import pallas as pl
from jax.experimental.pallas import tpu as pltpu
from jax.experimental.pallas import tpu_sc as plsc

N = 10000
E = 320000
R = 2
D = 128
H = 64
NP_ = 10016
AROWS = 2 * NP_
NTILES = 16
NCORES = 2
EB = 64
G = 8
NBLK = 320
NGRP = NBLK // G
EPT = NBLK * EB
EP = EPT * NTILES
RPT = AROWS // NTILES
HPT = NP_ // NTILES
BM = 1024
GRID = (NP_ + BM - 1) // BM


def _prep_body(dst_hbm, typ_hbm, z16_hbm, sidx_out, deg_out,
               dega, db0, tb0, ones, ssem, osem, wsem):
    c = lax.axis_index("c")
    s = lax.axis_index("s")
    row0 = s * RPT

    def _stage(grp):
        off = grp * G
        pltpu.async_copy(dst_hbm.at[s, pl.ds(off, G)], db0, ssem)
        pltpu.async_copy(typ_hbm.at[s, pl.ds(off, G)], tb0, ssem)

    def _stage_wait():
        dummy = dst_hbm.at[s, pl.ds(0, G)]
        pltpu.make_async_copy(dummy, db0, ssem).wait()
        pltpu.make_async_copy(dummy, tb0, ssem).wait()

    @pl.when(c == 0)
    def _():
        o16 = jnp.ones((16,), jnp.float32)

        @pl.loop(0, EB)
        def _fo(i):
            ones[i, pl.ds(0, 16)] = o16

        pltpu.sync_copy(z16_hbm, dega.at[pl.ds(row0, RPT)])
        _stage(0)

    plsc.subcore_barrier()

    @pl.when(c == 0)
    def _():
        @pl.loop(0, NGRP)
        def _g(grp):
            _stage_wait()

            @pl.loop(0, G * (EB // 16))
            def _(k):
                r = k // (EB // 16)
                q = (k % (EB // 16)) * 16
                db0[r, pl.ds(q, 16)] = (db0[r, pl.ds(q, 16)]
                                        + tb0[r, pl.ds(q, 16)] * NP_)

            pltpu.async_copy(db0, sidx_out.at[s, pl.ds(grp * G, G)], osem)
            for j in range(G):
                pltpu.async_copy(ones, dega.at[db0.at[j]], wsem, add=True)
            for j in range(G):
                pltpu.make_async_copy(z16_hbm.at[pl.ds(0, EB)], ones,
                                      wsem).wait()
            pltpu.make_async_copy(dst_hbm.at[s, pl.ds(0, G)], db0, osem).wait()

            @pl.when(grp + 1 < NGRP)
            def _():
                _stage(grp + 1)

    plsc.subcore_barrier()

    @pl.when(c == 0)
    def _():
        pltpu.sync_copy(
            dega.at[pl.ds(row0, RPT)],
            deg_out.at[s // (NTILES // R), pl.ds((s % (NTILES // R)) * RPT, RPT)])


def _make_prep():
    mesh = plsc.VectorSubcoreMesh(core_axis_name="c", subcore_axis_name="s",
                                  num_cores=NCORES, num_subcores=NTILES)
    return pl.kernel(
        _prep_body,
        out_type=(jax.ShapeDtypeStruct((NTILES, NBLK, EB), jnp.int32),
                  jax.ShapeDtypeStruct((R, NP_, 16), jnp.float32)),
        mesh=mesh,
        scratch_types=[
            pltpu.VMEM_SHARED((AROWS, 16), jnp.float32),
            pltpu.VMEM((G, EB), jnp.int32),
            pltpu.VMEM((G, EB), jnp.int32),
            pltpu.VMEM((EB, 16), jnp.float32),
            pltpu.SemaphoreType.DMA,
            pltpu.SemaphoreType.DMA,
            pltpu.SemaphoreType.DMA,
        ],
        compiler_params=pltpu.CompilerParams(use_tc_tiling_on_sc=False),
    )


def _layer_body(h_hbm, src_hbm, sidx_hbm, z64_hbm, s_out,
                acc, htab, sb0, db0, sb1, db1, rbuf0, rbuf1,
                gsem0, gsem1, ssem0, ssem1, wsem0, wsem1):
    c = lax.axis_index("c")
    s = lax.axis_index("s")
    rbufs = (rbuf0, rbuf1)
    gsems = (gsem0, gsem1)
    wsems = (wsem0, wsem1)
    sets = ((sb0, db0, ssem0), (sb1, db1, ssem1))

    def _stage(grp, si):
        sb, db, sem = sets[si]
        off = grp * G
        pltpu.async_copy(src_hbm.at[s, pl.ds(off, G)], sb, sem)
        pltpu.async_copy(sidx_hbm.at[s, pl.ds(off, G)], db, sem)

    def _stage_wait(si):
        sb, db, sem = sets[si]
        dummy = src_hbm.at[s, pl.ds(0, G)]
        pltpu.make_async_copy(dummy, sb, sem).wait()
        pltpu.make_async_copy(dummy, db, sem).wait()

    hrow0 = s * HPT
    pltpu.sync_copy(h_hbm.at[pl.ds(c * NP_ + hrow0, HPT)],
                    htab.at[pl.ds(hrow0, HPT)])
    row0 = s * RPT
    pltpu.sync_copy(z64_hbm, acc.at[pl.ds(row0, RPT)])
    _stage(0, 0)
    plsc.subcore_barrier()

    def _gwait(bi):
        pltpu.make_async_copy(h_hbm.at[pl.ds(0, EB)], rbufs[bi],
                              gsems[bi]).wait()

    def _wwait(bi):
        pltpu.make_async_copy(h_hbm.at[pl.ds(0, EB)], rbufs[bi],
                              wsems[bi]).wait()

    def _group(grp, si, stage_next, first, last):
        sb, db, _ = sets[si]
        _stage_wait(si)
        if stage_next:
            @pl.when(grp + 1 < NGRP)
            def _():
                _stage(grp + 1, 1 - si)
        pltpu.async_copy(htab.at[sb.at[0]], rbufs[0], gsems[0])
        for j in range(G):
            bi = j % 2
            if j + 1 < G:
                if not (first and j == 0):
                    _wwait(1 - bi)
                pltpu.async_copy(htab.at[sb.at[j + 1]], rbufs[1 - bi],
                                 gsems[1 - bi])
            _gwait(bi)
            pltpu.async_copy(rbufs[bi], acc.at[db.at[j]], wsems[bi], add=True)
        if last:
            _wwait(0)
            _wwait(1)
        else:
            _wwait(0)

    _group(0, 0, True, True, False)

    @pl.loop(0, NGRP // 2 - 1)
    def _pair(i):
        _group(2 * i + 1, 1, True, False, False)
        _group(2 * i + 2, 0, True, False, False)

    _group(NGRP - 1, 1, False, False, True)

    plsc.subcore_barrier()

    pltpu.sync_copy(
        acc.at[pl.ds(row0, RPT)],
        s_out.at[c, s // (NTILES // R), pl.ds((s % (NTILES // R)) * RPT, RPT)])


def _make_layer():
    mesh = plsc.VectorSubcoreMesh(core_axis_name="c", subcore_axis_name="s",
                                  num_cores=NCORES, num_subcores=NTILES)
    return pl.kernel(
        _layer_body,
        out_type=jax.ShapeDtypeStruct((NCORES, R, NP_, H), jnp.float32),
        mesh=mesh,
        scratch_types=[
            pltpu.VMEM_SHARED((AROWS, H), jnp.float32),
            pltpu.VMEM_SHARED((NP_, H), jnp.float32),
            pltpu.VMEM((G, EB), jnp.int32),
            pltpu.VMEM((G, EB), jnp.int32),
            pltpu.VMEM((G, EB), jnp.int32),
            pltpu.VMEM((G, EB), jnp.int32),
            pltpu.VMEM((EB, H), jnp.float32),
            pltpu.VMEM((EB, H), jnp.float32),
            pltpu.SemaphoreType.DMA,
            pltpu.SemaphoreType.DMA,
            pltpu.SemaphoreType.DMA,
            pltpu.SemaphoreType.DMA,
            pltpu.SemaphoreType.DMA,
            pltpu.SemaphoreType.DMA,
        ],
        compiler_params=pltpu.CompilerParams(use_tc_tiling_on_sc=False),
    )


_DOT = functools.partial(jnp.dot, preferred_element_type=jnp.float32)


def _encode_body(x_ref, wenc_ref, benc_ref, win_ref, bin_ref, ain_ref, out_ref):
    h = _DOT(x_ref[...], wenc_ref[...]) + benc_ref[...]
    h = jnp.where(h >= 0, h, 0.01 * h)
    h = _DOT(h, win_ref[...]) + bin_ref[...]
    h = jnp.where(h >= 0, h, ain_ref[...] * h)
    out_ref[0] = h[:, :H]
    out_ref[1] = h[:, H:]


def _combine_body(h_ref, s_ref, d_ref, wroot_ref, wrel_ref,
                  b_ref, wcls_ref, bcls_ref, out_ref, *, final):
    h = jnp.concatenate([h_ref[0], h_ref[1]], axis=1)
    s0 = jnp.concatenate([s_ref[0, 0], s_ref[1, 0]], axis=1)
    s1 = jnp.concatenate([s_ref[0, 1], s_ref[1, 1]], axis=1)
    inv0 = 1.0 / jnp.maximum(d_ref[0][:, :1], 1.0)
    inv1 = 1.0 / jnp.maximum(d_ref[1][:, :1], 1.0)
    o = (_DOT(h, wroot_ref[...]) + b_ref[...]
         + _DOT(s0 * inv0, wrel_ref[0]) + _DOT(s1 * inv1, wrel_ref[1]))
    if final:
        out_ref[...] = _DOT(o, wcls_ref[...]) + bcls_ref[...]
    else:
        out_ref[0] = o[:, :H]
        out_ref[1] = o[:, H:]


def _full(shape):
    return pl.BlockSpec(shape, lambda i: (0,) * len(shape))


_CAT_SPEC = pl.BlockSpec((2, BM, H), lambda i: (0, i, 0))


def _encode_call(x_p, wenc, benc, win, bin_, ain):
    return pl.pallas_call(
        _encode_body,
        grid=(GRID,),
        in_specs=[
            pl.BlockSpec((BM, 8), lambda i: (i, 0)),
            _full((8, D)), _full((1, D)), _full((D, D)), _full((1, D)),
            _full((1, D)),
        ],
        out_specs=_CAT_SPEC,
        out_shape=jax.ShapeDtypeStruct((2, NP_, H), jnp.float32),
    )(x_p, wenc, benc, win, bin_, ain)


def _combine_call(final, hcat, s_acc, deg, wroot, wrel, b, wcls, bcls):
    out_shape = (jax.ShapeDtypeStruct((NP_, D), jnp.float32) if final
                 else jax.ShapeDtypeStruct((2, NP_, H), jnp.float32))
    out_specs = (pl.BlockSpec((BM, D), lambda i: (i, 0)) if final
                 else _CAT_SPEC)
    return pl.pallas_call(
        functools.partial(_combine_body, final=final),
        grid=(GRID,),
        in_specs=[
            _CAT_SPEC,
            pl.BlockSpec((2, 2, BM, H), lambda i: (0, 0, i, 0)),
            pl.BlockSpec((2, BM, 16), lambda i: (0, i, 0)),
            _full((D, D)), _full((R, D, D)), _full((1, D)),
            _full((D, D)), _full((1, D)),
        ],
        out_specs=out_specs,
        out_shape=out_shape,
    )(hcat, s_acc, deg, wroot, wrel, b, wcls, bcls)


_sc_cache = {}


def _sc_prep(*args):
    if "prep" not in _sc_cache:
        _sc_cache["prep"] = _make_prep()
    return _sc_cache["prep"](*args)


def _sc_layer(*args):
    if "layer" not in _sc_cache:
        _sc_cache["layer"] = _make_layer()
    return _sc_cache["layer"](*args)


def kernel(x, edge_index, edge_type, W_np, b_np, W_cp, b_cp, W_in, b_in, a_in,
           W_rel1, W_root1, b1, W_rel2, W_root2, b2, W_cls, b_cls):
    f32 = jnp.float32
    x_p = jnp.pad(x, ((0, NP_ - N), (0, 0)))
    wenc = jnp.zeros((8, D), f32).at[:5, :H].set(W_np).at[5:, H:].set(W_cp)
    benc = jnp.concatenate([b_np, b_cp]).reshape(1, D)
    src = jnp.pad(edge_index[0], (0, EP - E)).reshape(NTILES, NBLK, EB)
    dst = jnp.pad(edge_index[1], (0, EP - E),
                  constant_values=N).reshape(NTILES, NBLK, EB)
    typ = jnp.pad(edge_type, (0, EP - E)).reshape(NTILES, NBLK, EB)
    z16 = jnp.zeros((RPT, 16), f32)
    z64 = jnp.zeros((RPT, H), f32)

    sidx, deg = _sc_prep(dst, typ, z16)

    h0cat = _encode_call(x_p, wenc, benc, W_in, b_in.reshape(1, D),
                         a_in.reshape(1, D))

    s1 = _sc_layer(h0cat.reshape(NCORES * NP_, H), src, sidx, z64)
    h1cat = _combine_call(False, h0cat, s1, deg, W_root1, W_rel1,
                          b1.reshape(1, D), W_cls, b_cls.reshape(1, D))

    s2 = _sc_layer(h1cat.reshape(NCORES * NP_, H), src, sidx, z64)
    out = _combine_call(True, h1cat, s2, deg, W_root2, W_rel2,
                        b2.reshape(1, D), W_cls, b_cls.reshape(1, D))
    return out[:N]
